# baseline (speedup 1.0000x reference)
import functools

import jax
import jax.numpy as jnp
from jax import lax
from jax.experimental import pallas as pl
from jax.experimental.pallas import tpu as pltpu
from jax.experimental.pallas import tpu_sc as plsc

N = 10000
E = 320000
D1 = 128
D2 = 256
G = 64

NC = 2
NS = 16
CHUNK = 80
NPAD = 10240
ROWS_PER_TILE = NPAD // NS
ZROWS = 32

def _vmesh():
    return plsc.VectorSubcoreMesh(core_axis_name="c", subcore_axis_name="s")


def _zero_fill(buf):
    @pl.loop(0, CHUNK)
    def _(i):
        for j in range(8):
            buf[i, pl.ds(j * 16, 16)] = jnp.zeros((16,), jnp.float32)


NSETS = 3


def _scatter_chunks(table, src_flat, dst_flat, base, acc, isrc, idst, rows,
                    sem_i, sem_g, sem_s, nchunks):
    def idx_load(c, j):
        off = base + c * CHUNK
        pltpu.async_copy(src_flat.at[pl.ds(off, CHUNK)], isrc[j], sem_i[j])
        pltpu.async_copy(dst_flat.at[pl.ds(off, CHUNK)], idst[j], sem_i[j])

    def idx_wait(j):
        pltpu.make_async_copy(src_flat.at[pl.ds(0, CHUNK)], isrc[j],
                              sem_i[j]).wait()
        pltpu.make_async_copy(dst_flat.at[pl.ds(0, CHUNK)], idst[j],
                              sem_i[j]).wait()

    def gather_wait(j):
        pltpu.make_async_copy(table.at[isrc[j]], rows[j], sem_g[j]).wait()

    def scatter_start(j):
        pltpu.async_copy(rows[j], acc.at[idst[j]], sem_s[j], add=True)

    def scatter_wait(j):
        pltpu.make_async_copy(rows[j], acc.at[idst[j]], sem_s[j]).wait()

    nrounds = nchunks // NSETS
    tail = nchunks % NSETS

    for j in range(NSETS):
        idx_load(j, j)
    for j in range(NSETS):
        idx_wait(j)
        pltpu.async_copy(table.at[isrc[j]], rows[j], sem_g[j])

    @pl.loop(0, nrounds)
    def _(r):
        c0 = r * NSETS
        for j in range(NSETS):
            gather_wait(j)
            scatter_start(j)
        for j in range(NSETS):
            scatter_wait(j)

            @pl.when(c0 + NSETS + j < nchunks)
            def _():
                idx_load(c0 + NSETS + j, j)
        for j in range(NSETS):

            @pl.when(c0 + NSETS + j < nchunks)
            def _():
                idx_wait(j)
                pltpu.async_copy(table.at[isrc[j]], rows[j], sem_g[j])

    for j in range(tail):
        gather_wait(j)
        scatter_start(j)
    for j in range(tail):
        scatter_wait(j)


def _sc_prologue(acc, rows0, sid):
    _zero_fill(rows0)
    row0 = sid * ROWS_PER_TILE
    for kk in range(ROWS_PER_TILE // CHUNK):
        pltpu.sync_copy(rows0, acc.at[pl.ds(row0 + kk * CHUNK, CHUNK)])


def _sc_epilogue(acc, rows0, out, cid, sid):
    plsc.subcore_barrier()
    row0 = sid * ROWS_PER_TILE
    for k in range(ROWS_PER_TILE // CHUNK):
        sl = pl.ds(row0 + k * CHUNK, CHUNK)
        pltpu.sync_copy(acc.at[sl], rows0)
        pltpu.sync_copy(rows0, out.at[cid, sl])


def _sc_scratch():
    t = []
    for _ in range(NSETS):
        t.append(pltpu.VMEM((CHUNK,), jnp.int32))
    for _ in range(NSETS):
        t.append(pltpu.VMEM((CHUNK,), jnp.int32))
    for _ in range(NSETS):
        t.append(pltpu.VMEM((CHUNK, 128), jnp.float32))
    t.append(pltpu.VMEM_SHARED((NPAD, 128), jnp.float32))
    for _ in range(3 * NSETS):
        t.append(pltpu.SemaphoreType.DMA)
    return t


NCH_P = E // (NC * NS * CHUNK)
NCH_C = E // (NS * CHUNK)


@jax.jit
def _sc_pass_partial(table, src, dst):
    @functools.partial(
        pl.kernel, mesh=_vmesh(),
        out_type=jax.ShapeDtypeStruct((NC, NPAD, 128), jnp.float32),
        scratch_types=_sc_scratch(),
    )
    def k(table_h, src_h, dst_h, out_h, *scr):
        isrc = scr[0:NSETS]
        idst = scr[NSETS:2 * NSETS]
        rows = scr[2 * NSETS:3 * NSETS]
        acc = scr[3 * NSETS]
        sem_i = scr[3 * NSETS + 1:3 * NSETS + 1 + NSETS]
        sem_g = scr[3 * NSETS + 1 + NSETS:3 * NSETS + 1 + 2 * NSETS]
        sem_s = scr[3 * NSETS + 1 + 2 * NSETS:3 * NSETS + 1 + 3 * NSETS]
        cid = lax.axis_index("c")
        sid = lax.axis_index("s")
        w = cid * NS + sid
        _sc_prologue(acc, rows[0], sid)
        plsc.subcore_barrier()
        _scatter_chunks(table_h, src_h, dst_h, w * (E // (NC * NS)), acc,
                        isrc, idst, rows, sem_i, sem_g, sem_s, NCH_P)
        _sc_epilogue(acc, rows[0], out_h, cid, sid)

    return k(table, src, dst)


@jax.jit
def _sc_pass_colsplit(table_lo, table_hi, src, dst):
    @functools.partial(
        pl.kernel, mesh=_vmesh(),
        out_type=jax.ShapeDtypeStruct((NC, NPAD, 128), jnp.float32),
        scratch_types=_sc_scratch(),
    )
    def k(lo_h, hi_h, src_h, dst_h, out_h, *scr):
        isrc = scr[0:NSETS]
        idst = scr[NSETS:2 * NSETS]
        rows = scr[2 * NSETS:3 * NSETS]
        acc = scr[3 * NSETS]
        sem_i = scr[3 * NSETS + 1:3 * NSETS + 1 + NSETS]
        sem_g = scr[3 * NSETS + 1 + NSETS:3 * NSETS + 1 + 2 * NSETS]
        sem_s = scr[3 * NSETS + 1 + 2 * NSETS:3 * NSETS + 1 + 3 * NSETS]
        cid = lax.axis_index("c")
        sid = lax.axis_index("s")
        _sc_prologue(acc, rows[0], sid)
        plsc.subcore_barrier()
        base = sid * (E // NS)

        @pl.when(cid == 0)
        def _():
            _scatter_chunks(lo_h, src_h, dst_h, base, acc, isrc, idst,
                            rows, sem_i, sem_g, sem_s, NCH_C)

        @pl.when(cid == 1)
        def _():
            _scatter_chunks(hi_h, src_h, dst_h, base, acc, isrc, idst,
                            rows, sem_i, sem_g, sem_s, NCH_C)

        _sc_epilogue(acc, rows[0], out_h, cid, sid)

    return k(table_lo, table_hi, src, dst)


RBLK = 2000

_DOT = dict(preferred_element_type=jnp.float32, precision=lax.Precision.HIGHEST)


def _gru_body(*refs, d, relu, nh, nout):
    it = iter(refs)
    sa_ref, sb_ref = next(it), next(it)
    h_refs = [next(it) for _ in range(nh)]
    wa_ref, wb_ref = next(it), next(it)
    whh_refs = [next(it) for _ in range(nh)]
    bih_ref, bhh_ref = next(it), next(it)
    o_refs = [next(it) for _ in range(nout)]

    gi = (jnp.dot(sa_ref[...], wa_ref[...], **_DOT)
          + jnp.dot(sb_ref[...], wb_ref[...], **_DOT) + bih_ref[...])
    gh = bhh_ref[...].astype(jnp.float32)
    for h_ref, w_ref in zip(h_refs, whh_refs):
        gh = gh + jnp.dot(h_ref[...], w_ref[...], **_DOT)
    r = jax.nn.sigmoid(gi[:, :d] + gh[:, :d])
    z = jax.nn.sigmoid(gi[:, d:2 * d] + gh[:, d:2 * d])
    nn = jnp.tanh(gi[:, 2 * d:] + r * gh[:, 2 * d:])
    h = jnp.concatenate([h_ref[...] for h_ref in h_refs], axis=1)
    if h.shape[1] < d:
        h = jnp.concatenate(
            [h, jnp.zeros((h.shape[0], d - h.shape[1]), h.dtype)], axis=1)
    out = (1.0 - z) * nn + z * h
    if relu:
        out = jnp.maximum(out, 0.0)
    if nout == 1:
        o_refs[0][...] = out
    else:
        for i, o_ref in enumerate(o_refs):
            o_ref[...] = out[:, i * 128:(i + 1) * 128]


def _tc_gru(sa, sb, hs, wca, wcb, whhs, bih, bhh, d, relu, nout):
    grid = N // RBLK
    nh = len(hs)
    in_specs = [
        pl.BlockSpec((RBLK, 128), lambda i: (i, 0)),
        pl.BlockSpec((RBLK, 128), lambda i: (i, 0)),
    ]
    for h in hs:
        in_specs.append(pl.BlockSpec((RBLK, h.shape[1]), lambda i: (i, 0)))
    in_specs.append(pl.BlockSpec((128, 3 * d), lambda i: (0, 0)))
    in_specs.append(pl.BlockSpec((128, 3 * d), lambda i: (0, 0)))
    for w in whhs:
        in_specs.append(pl.BlockSpec(w.shape, lambda i: (0, 0)))
    in_specs.append(pl.BlockSpec((1, 3 * d), lambda i: (0, 0)))
    in_specs.append(pl.BlockSpec((1, 3 * d), lambda i: (0, 0)))
    if nout == 1:
        out_specs = pl.BlockSpec((RBLK, d), lambda i: (i, 0))
        out_shape = jax.ShapeDtypeStruct((N, d), jnp.float32)
    else:
        out_specs = [pl.BlockSpec((RBLK, 128), lambda i: (i, 0))
                     for _ in range(nout)]
        out_shape = [jax.ShapeDtypeStruct((N, 128), jnp.float32)
                     for _ in range(nout)]
    return pl.pallas_call(
        functools.partial(_gru_body, d=d, relu=relu, nh=nh, nout=nout),
        grid=(grid,),
        in_specs=in_specs,
        out_specs=out_specs,
        out_shape=out_shape,
    )(sa, sb, *hs, wca, wcb, *whhs, bih, bhh)


SEG_CS = 512


def _segmax_body(xlo_ref, xhi_ref, b_ref, wf_ref, bf_ref, o_ref):
    b = b_ref[...]
    row_iota = lax.broadcasted_iota(jnp.int32, (G, 1), 0)

    def body(g, segs):
        start = jnp.sum((b < g).astype(jnp.int32))
        end = jnp.sum((b <= g).astype(jnp.int32))
        base = (start // 8) * 8
        nch = (end - base + SEG_CS - 1) // SEG_CS

        def chunk(j, acc):
            off = jnp.minimum(base + j * SEG_CS, N - SEG_CS)
            rows = off + lax.broadcasted_iota(jnp.int32, (SEG_CS, 1), 0)
            mask = (rows >= start) & (rows < end)
            v = jnp.concatenate(
                [xlo_ref[pl.ds(off, SEG_CS), :],
                 xhi_ref[pl.ds(off, SEG_CS), :]], axis=1)
            v = jnp.where(mask, v, -jnp.inf)
            return jnp.maximum(acc, jnp.max(v, axis=0, keepdims=True))

        seg = lax.fori_loop(0, nch, chunk,
                            jnp.full((1, D2), -jnp.inf, jnp.float32))
        return jnp.where(row_iota == g, seg, segs)

    segs = lax.fori_loop(0, G, body,
                         jnp.full((G, D2), -jnp.inf, jnp.float32))
    o_ref[...] = jnp.dot(segs, wf_ref[...], **_DOT) + bf_ref[...]


def _tc_segmax_head(xlo, xhi, batch2d, wf_pad, bf_pad):
    return pl.pallas_call(
        _segmax_body,
        grid=(1,),
        in_specs=[
            pl.BlockSpec((N, 128), lambda g: (0, 0)),
            pl.BlockSpec((N, 128), lambda g: (0, 0)),
            pl.BlockSpec((N, 1), lambda g: (0, 0)),
            pl.BlockSpec((D2, 128), lambda g: (0, 0)),
            pl.BlockSpec((1, 128), lambda g: (0, 0)),
        ],
        out_specs=pl.BlockSpec((G, 128), lambda g: (0, 0)),
        out_shape=jax.ShapeDtypeStruct((G, 128), jnp.float32),
    )(xlo, xhi, batch2d, wf_pad, bf_pad)


def kernel(x, edge_index, batch, weight1, Wih1, Whh1, bih1, bhh1,
           weight2, Wih2, Whh2, bih2, bhh2, Wf, bf):
    src = edge_index[0].astype(jnp.int32)
    dst = edge_index[1].astype(jnp.int32)

    wc1_0 = weight1[0] @ Wih1.T
    wc1_1 = weight1[1] @ Wih1.T
    wc2_0 = weight2[0][:128, :] @ Wih2.T
    wc2_1 = weight2[1] @ Wih2.T
    whhT1 = Whh1.T
    whhT2 = Whh2.T
    bih1r = bih1.reshape(1, -1)
    bhh1r = bhh1.reshape(1, -1)
    bih2r = bih2.reshape(1, -1)
    bhh2r = bhh2.reshape(1, -1)

    s1 = _sc_pass_partial(x, src, dst)
    x1 = _tc_gru(s1[0], s1[1], [x], wc1_0, wc1_0, [whhT1], bih1r, bhh1r,
                 D1, False, 1)
    s2 = _sc_pass_partial(x1, src, dst)
    y = _tc_gru(s2[0], s2[1], [x1], wc1_1, wc1_1, [whhT1], bih1r, bhh1r,
                D1, True, 1)

    s3 = _sc_pass_partial(y, src, dst)
    x3lo, x3hi = _tc_gru(s3[0], s3[1], [y], wc2_0, wc2_0, [whhT2[:128, :]],
                         bih2r, bhh2r, D2, False, 2)

    s4 = _sc_pass_colsplit(x3lo, x3hi, src, dst)
    x4lo, x4hi = _tc_gru(s4[0], s4[1], [x3lo, x3hi], wc2_1[:128, :],
                         wc2_1[128:, :], [whhT2[:128, :], whhT2[128:, :]],
                         bih2r, bhh2r, D2, False, 2)

    batch2d = batch.astype(jnp.int32).reshape(N, 1)
    wf_pad = jnp.zeros((D2, 128), jnp.float32).at[:, :6].set(Wf.T)
    bf_pad = jnp.zeros((1, 128), jnp.float32).at[0, :6].set(bf)
    out = _tc_segmax_head(x4lo, x4hi, batch2d, wf_pad, bf_pad)
    return out[:, :6]


# speedup vs baseline: 1.0850x; 1.0850x over previous
import functools

import jax
import jax.numpy as jnp
from jax import lax
from jax.experimental import pallas as pl
from jax.experimental.pallas import tpu as pltpu
from jax.experimental.pallas import tpu_sc as plsc

N = 10000
E = 320000
D1 = 128
D2 = 256
G = 64

NC = 2
NS = 16
CHUNK = 80
NPAD = 10240
ROWS_PER_TILE = NPAD // NS
ZROWS = 32

def _vmesh():
    return plsc.VectorSubcoreMesh(core_axis_name="c", subcore_axis_name="s")


def _zero_fill(buf):
    @pl.loop(0, CHUNK)
    def _(i):
        for j in range(8):
            buf[i, pl.ds(j * 16, 16)] = jnp.zeros((16,), jnp.float32)


NSETS = 3


def _scatter_chunks(table, src_flat, dst_flat, base, acc, isrc, idst, rows,
                    sem_i, sem_g, sem_s, nchunks):
    def idx_load(c, j):
        off = base + c * CHUNK
        pltpu.async_copy(src_flat.at[pl.ds(off, CHUNK)], isrc[j], sem_i[j])
        pltpu.async_copy(dst_flat.at[pl.ds(off, CHUNK)], idst[j], sem_i[j])

    def idx_wait(j):
        pltpu.make_async_copy(src_flat.at[pl.ds(0, CHUNK)], isrc[j],
                              sem_i[j]).wait()
        pltpu.make_async_copy(dst_flat.at[pl.ds(0, CHUNK)], idst[j],
                              sem_i[j]).wait()

    def gather_wait(j):
        pltpu.make_async_copy(table.at[isrc[j]], rows[j], sem_g[j]).wait()

    def scatter_start(j):
        pltpu.async_copy(rows[j], acc.at[idst[j]], sem_s[j], add=True)

    def scatter_wait(j):
        pltpu.make_async_copy(rows[j], acc.at[idst[j]], sem_s[j]).wait()

    nrounds = nchunks // NSETS
    tail = nchunks % NSETS

    for j in range(NSETS):
        idx_load(j, j)
    for j in range(NSETS):
        idx_wait(j)
        pltpu.async_copy(table.at[isrc[j]], rows[j], sem_g[j])

    @pl.loop(0, nrounds)
    def _(r):
        c0 = r * NSETS
        for j in range(NSETS):
            gather_wait(j)
            scatter_start(j)
        for j in range(NSETS):
            scatter_wait(j)

            @pl.when(c0 + NSETS + j < nchunks)
            def _():
                idx_load(c0 + NSETS + j, j)
        for j in range(NSETS):

            @pl.when(c0 + NSETS + j < nchunks)
            def _():
                idx_wait(j)
                pltpu.async_copy(table.at[isrc[j]], rows[j], sem_g[j])

    for j in range(tail):
        gather_wait(j)
        scatter_start(j)
    for j in range(tail):
        scatter_wait(j)


def _sc_prologue(acc, rows0, sid):
    _zero_fill(rows0)
    row0 = sid * ROWS_PER_TILE
    for kk in range(ROWS_PER_TILE // CHUNK):
        pltpu.sync_copy(rows0, acc.at[pl.ds(row0 + kk * CHUNK, CHUNK)])


def _sc_epilogue(acc, rows0, out, cid, sid):
    plsc.subcore_barrier()
    row0 = sid * ROWS_PER_TILE
    for k in range(ROWS_PER_TILE // CHUNK):
        sl = pl.ds(row0 + k * CHUNK, CHUNK)
        pltpu.sync_copy(acc.at[sl], rows0)
        pltpu.sync_copy(rows0, out.at[cid, sl])


def _sc_scratch():
    t = []
    for _ in range(NSETS):
        t.append(pltpu.VMEM((CHUNK,), jnp.int32))
    for _ in range(NSETS):
        t.append(pltpu.VMEM((CHUNK,), jnp.int32))
    for _ in range(NSETS):
        t.append(pltpu.VMEM((CHUNK, 128), jnp.float32))
    t.append(pltpu.VMEM_SHARED((NPAD, 128), jnp.float32))
    for _ in range(3 * NSETS):
        t.append(pltpu.SemaphoreType.DMA)
    return t


NCH_P = E // (NC * NS * CHUNK)
NCH_C = E // (NS * CHUNK)


@jax.jit
def _sc_pass_partial(table, src, dst):
    @functools.partial(
        pl.kernel, mesh=_vmesh(),
        out_type=jax.ShapeDtypeStruct((NC, NPAD, 128), jnp.float32),
        scratch_types=_sc_scratch(),
    )
    def k(table_h, src_h, dst_h, out_h, *scr):
        isrc = scr[0:NSETS]
        idst = scr[NSETS:2 * NSETS]
        rows = scr[2 * NSETS:3 * NSETS]
        acc = scr[3 * NSETS]
        sem_i = scr[3 * NSETS + 1:3 * NSETS + 1 + NSETS]
        sem_g = scr[3 * NSETS + 1 + NSETS:3 * NSETS + 1 + 2 * NSETS]
        sem_s = scr[3 * NSETS + 1 + 2 * NSETS:3 * NSETS + 1 + 3 * NSETS]
        cid = lax.axis_index("c")
        sid = lax.axis_index("s")
        w = cid * NS + sid
        _sc_prologue(acc, rows[0], sid)
        plsc.subcore_barrier()
        _scatter_chunks(table_h, src_h, dst_h, w * (E // (NC * NS)), acc,
                        isrc, idst, rows, sem_i, sem_g, sem_s, NCH_P)
        _sc_epilogue(acc, rows[0], out_h, cid, sid)

    return k(table, src, dst)


@jax.jit
def _sc_pass_colsplit(table_lo, table_hi, src, dst):
    @functools.partial(
        pl.kernel, mesh=_vmesh(),
        out_type=jax.ShapeDtypeStruct((NC, NPAD, 128), jnp.float32),
        scratch_types=_sc_scratch(),
    )
    def k(lo_h, hi_h, src_h, dst_h, out_h, *scr):
        isrc = scr[0:NSETS]
        idst = scr[NSETS:2 * NSETS]
        rows = scr[2 * NSETS:3 * NSETS]
        acc = scr[3 * NSETS]
        sem_i = scr[3 * NSETS + 1:3 * NSETS + 1 + NSETS]
        sem_g = scr[3 * NSETS + 1 + NSETS:3 * NSETS + 1 + 2 * NSETS]
        sem_s = scr[3 * NSETS + 1 + 2 * NSETS:3 * NSETS + 1 + 3 * NSETS]
        cid = lax.axis_index("c")
        sid = lax.axis_index("s")
        _sc_prologue(acc, rows[0], sid)
        plsc.subcore_barrier()
        base = sid * (E // NS)

        @pl.when(cid == 0)
        def _():
            _scatter_chunks(lo_h, src_h, dst_h, base, acc, isrc, idst,
                            rows, sem_i, sem_g, sem_s, NCH_C)

        @pl.when(cid == 1)
        def _():
            _scatter_chunks(hi_h, src_h, dst_h, base, acc, isrc, idst,
                            rows, sem_i, sem_g, sem_s, NCH_C)

        _sc_epilogue(acc, rows[0], out_h, cid, sid)

    return k(table_lo, table_hi, src, dst)


RBLK = 2000

_DOT = dict(preferred_element_type=jnp.float32, precision=lax.Precision.HIGHEST)


def _matvec_body(*refs, nh):
    it = iter(refs)
    h_refs = [next(it) for _ in range(nh)]
    w_refs = [next(it) for _ in range(nh)]
    b_ref = next(it)
    o_ref = next(it)
    acc = b_ref[...].astype(jnp.float32)
    for h_ref, w_ref in zip(h_refs, w_refs):
        acc = acc + jnp.dot(h_ref[...], w_ref[...], **_DOT)
    o_ref[...] = acc


def _tc_matvec(hs, ws, b):
    nh = len(hs)
    m = ws[0].shape[1]
    in_specs = [pl.BlockSpec((RBLK, h.shape[1]), lambda i: (i, 0))
                for h in hs]
    in_specs += [pl.BlockSpec(w.shape, lambda i: (0, 0)) for w in ws]
    in_specs.append(pl.BlockSpec((1, m), lambda i: (0, 0)))
    return pl.pallas_call(
        functools.partial(_matvec_body, nh=nh),
        grid=(N // RBLK,),
        in_specs=in_specs,
        out_specs=pl.BlockSpec((RBLK, m), lambda i: (i, 0)),
        out_shape=jax.ShapeDtypeStruct((N, m), jnp.float32),
    )(*hs, *ws, b)


def _gru_body(*refs, d, relu, nh, nout):
    it = iter(refs)
    sa_ref, sb_ref, gh_ref = next(it), next(it), next(it)
    h_refs = [next(it) for _ in range(nh)]
    wa_ref, wb_ref = next(it), next(it)
    bih_ref = next(it)
    o_refs = [next(it) for _ in range(nout)]

    gi = (jnp.dot(sa_ref[...], wa_ref[...], **_DOT)
          + jnp.dot(sb_ref[...], wb_ref[...], **_DOT) + bih_ref[...])
    gh = gh_ref[...]
    r = jax.nn.sigmoid(gi[:, :d] + gh[:, :d])
    z = jax.nn.sigmoid(gi[:, d:2 * d] + gh[:, d:2 * d])
    nn = jnp.tanh(gi[:, 2 * d:] + r * gh[:, 2 * d:])
    h = jnp.concatenate([h_ref[...] for h_ref in h_refs], axis=1)
    if h.shape[1] < d:
        h = jnp.concatenate(
            [h, jnp.zeros((h.shape[0], d - h.shape[1]), h.dtype)], axis=1)
    out = (1.0 - z) * nn + z * h
    if relu:
        out = jnp.maximum(out, 0.0)
    if nout == 1:
        o_refs[0][...] = out
    else:
        for i, o_ref in enumerate(o_refs):
            o_ref[...] = out[:, i * 128:(i + 1) * 128]


def _tc_gru(sa, sb, gh, hs, wca, wcb, bih, d, relu, nout):
    nh = len(hs)
    in_specs = [
        pl.BlockSpec((RBLK, 128), lambda i: (i, 0)),
        pl.BlockSpec((RBLK, 128), lambda i: (i, 0)),
        pl.BlockSpec((RBLK, 3 * d), lambda i: (i, 0)),
    ]
    for h in hs:
        in_specs.append(pl.BlockSpec((RBLK, h.shape[1]), lambda i: (i, 0)))
    in_specs.append(pl.BlockSpec((128, 3 * d), lambda i: (0, 0)))
    in_specs.append(pl.BlockSpec((128, 3 * d), lambda i: (0, 0)))
    in_specs.append(pl.BlockSpec((1, 3 * d), lambda i: (0, 0)))
    if nout == 1:
        out_specs = pl.BlockSpec((RBLK, d), lambda i: (i, 0))
        out_shape = jax.ShapeDtypeStruct((N, d), jnp.float32)
    else:
        out_specs = [pl.BlockSpec((RBLK, 128), lambda i: (i, 0))
                     for _ in range(nout)]
        out_shape = [jax.ShapeDtypeStruct((N, 128), jnp.float32)
                     for _ in range(nout)]
    return pl.pallas_call(
        functools.partial(_gru_body, d=d, relu=relu, nh=nh, nout=nout),
        grid=(N // RBLK,),
        in_specs=in_specs,
        out_specs=out_specs,
        out_shape=out_shape,
    )(sa, sb, gh, *hs, wca, wcb, bih)


SEG_CS = 512


def _segmax_body(xlo_ref, xhi_ref, b_ref, wf_ref, bf_ref, o_ref):
    b = b_ref[...]
    row_iota = lax.broadcasted_iota(jnp.int32, (G, 1), 0)

    def body(g, segs):
        start = jnp.sum((b < g).astype(jnp.int32))
        end = jnp.sum((b <= g).astype(jnp.int32))
        base = (start // 8) * 8
        nch = (end - base + SEG_CS - 1) // SEG_CS

        def chunk(j, acc):
            off = jnp.minimum(base + j * SEG_CS, N - SEG_CS)
            rows = off + lax.broadcasted_iota(jnp.int32, (SEG_CS, 1), 0)
            mask = (rows >= start) & (rows < end)
            v = jnp.concatenate(
                [xlo_ref[pl.ds(off, SEG_CS), :],
                 xhi_ref[pl.ds(off, SEG_CS), :]], axis=1)
            v = jnp.where(mask, v, -jnp.inf)
            return jnp.maximum(acc, jnp.max(v, axis=0, keepdims=True))

        seg = lax.fori_loop(0, nch, chunk,
                            jnp.full((1, D2), -jnp.inf, jnp.float32))
        return jnp.where(row_iota == g, seg, segs)

    segs = lax.fori_loop(0, G, body,
                         jnp.full((G, D2), -jnp.inf, jnp.float32))
    o_ref[...] = jnp.dot(segs, wf_ref[...], **_DOT) + bf_ref[...]


def _tc_segmax_head(xlo, xhi, batch2d, wf_pad, bf_pad):
    return pl.pallas_call(
        _segmax_body,
        grid=(1,),
        in_specs=[
            pl.BlockSpec((N, 128), lambda g: (0, 0)),
            pl.BlockSpec((N, 128), lambda g: (0, 0)),
            pl.BlockSpec((N, 1), lambda g: (0, 0)),
            pl.BlockSpec((D2, 128), lambda g: (0, 0)),
            pl.BlockSpec((1, 128), lambda g: (0, 0)),
        ],
        out_specs=pl.BlockSpec((G, 128), lambda g: (0, 0)),
        out_shape=jax.ShapeDtypeStruct((G, 128), jnp.float32),
    )(xlo, xhi, batch2d, wf_pad, bf_pad)


def kernel(x, edge_index, batch, weight1, Wih1, Whh1, bih1, bhh1,
           weight2, Wih2, Whh2, bih2, bhh2, Wf, bf):
    src = edge_index[0].astype(jnp.int32)
    dst = edge_index[1].astype(jnp.int32)

    wc1_0 = weight1[0] @ Wih1.T
    wc1_1 = weight1[1] @ Wih1.T
    wc2_0 = weight2[0][:128, :] @ Wih2.T
    wc2_1 = weight2[1] @ Wih2.T
    whhT1 = Whh1.T
    whhT2 = Whh2.T
    bih1r = bih1.reshape(1, -1)
    bhh1r = bhh1.reshape(1, -1)
    bih2r = bih2.reshape(1, -1)
    bhh2r = bhh2.reshape(1, -1)

    s1 = _sc_pass_partial(x, src, dst)
    gh1 = _tc_matvec([x], [whhT1], bhh1r)
    x1 = _tc_gru(s1[0], s1[1], gh1, [x], wc1_0, wc1_0, bih1r, D1, False, 1)

    s2 = _sc_pass_partial(x1, src, dst)
    gh2 = _tc_matvec([x1], [whhT1], bhh1r)
    y = _tc_gru(s2[0], s2[1], gh2, [x1], wc1_1, wc1_1, bih1r, D1, True, 1)

    s3 = _sc_pass_partial(y, src, dst)
    gh3 = _tc_matvec([y], [whhT2[:128, :]], bhh2r)
    x3lo, x3hi = _tc_gru(s3[0], s3[1], gh3, [y], wc2_0, wc2_0, bih2r,
                         D2, False, 2)

    s4 = _sc_pass_colsplit(x3lo, x3hi, src, dst)
    gh4 = _tc_matvec([x3lo, x3hi], [whhT2[:128, :], whhT2[128:, :]], bhh2r)
    x4lo, x4hi = _tc_gru(s4[0], s4[1], gh4, [x3lo, x3hi], wc2_1[:128, :],
                         wc2_1[128:, :], bih2r, D2, False, 2)

    batch2d = batch.astype(jnp.int32).reshape(N, 1)
    wf_pad = jnp.zeros((D2, 128), jnp.float32).at[:, :6].set(Wf.T)
    bf_pad = jnp.zeros((1, 128), jnp.float32).at[0, :6].set(bf)
    out = _tc_segmax_head(x4lo, x4hi, batch2d, wf_pad, bf_pad)
    return out[:, :6]


# speedup vs baseline: 1.1496x; 1.0596x over previous
import functools

import jax
import jax.numpy as jnp
from jax import lax
from jax.experimental import pallas as pl
from jax.experimental.pallas import tpu as pltpu
from jax.experimental.pallas import tpu_sc as plsc

N = 10000
E = 320000
D1 = 128
D2 = 256
G = 64

NC = 2
NS = 16
CHUNK = 80
NPAD = 10240
ROWS_PER_TILE = NPAD // NS
ZROWS = 32

def _vmesh():
    return plsc.VectorSubcoreMesh(core_axis_name="c", subcore_axis_name="s")


def _zero_fill(buf):
    @pl.loop(0, CHUNK)
    def _(i):
        for j in range(8):
            buf[i, pl.ds(j * 16, 16)] = jnp.zeros((16,), jnp.float32)


NSETS = 3


def _scatter_chunks(table, src_flat, dst_flat, base, acc, isrc, idst, rows,
                    sem_i, sem_g, sem_s, nchunks):
    def idx_load(c, j):
        off = base + c * CHUNK
        pltpu.async_copy(src_flat.at[pl.ds(off, CHUNK)], isrc[j], sem_i[j])
        pltpu.async_copy(dst_flat.at[pl.ds(off, CHUNK)], idst[j], sem_i[j])

    def idx_wait(j):
        pltpu.make_async_copy(src_flat.at[pl.ds(0, CHUNK)], isrc[j],
                              sem_i[j]).wait()
        pltpu.make_async_copy(dst_flat.at[pl.ds(0, CHUNK)], idst[j],
                              sem_i[j]).wait()

    def gather_wait(j):
        pltpu.make_async_copy(table.at[isrc[j]], rows[j], sem_g[j]).wait()

    def scatter_start(j):
        pltpu.async_copy(rows[j], acc.at[idst[j]], sem_s[j], add=True)

    def scatter_wait(j):
        pltpu.make_async_copy(rows[j], acc.at[idst[j]], sem_s[j]).wait()

    nrounds = nchunks // NSETS
    tail = nchunks % NSETS

    for j in range(NSETS):
        idx_load(j, j)
    for j in range(NSETS):
        idx_wait(j)
        pltpu.async_copy(table.at[isrc[j]], rows[j], sem_g[j])

    @pl.loop(0, nrounds)
    def _(r):
        c0 = r * NSETS
        for j in range(NSETS):
            gather_wait(j)
            scatter_start(j)
        for j in range(NSETS):
            scatter_wait(j)

            @pl.when(c0 + NSETS + j < nchunks)
            def _():
                idx_load(c0 + NSETS + j, j)
        for j in range(NSETS):

            @pl.when(c0 + NSETS + j < nchunks)
            def _():
                idx_wait(j)
                pltpu.async_copy(table.at[isrc[j]], rows[j], sem_g[j])

    for j in range(tail):
        gather_wait(j)
        scatter_start(j)
    for j in range(tail):
        scatter_wait(j)


def _sc_prologue(acc, rows0, sid):
    _zero_fill(rows0)
    row0 = sid * ROWS_PER_TILE
    for kk in range(ROWS_PER_TILE // CHUNK):
        pltpu.sync_copy(rows0, acc.at[pl.ds(row0 + kk * CHUNK, CHUNK)])


def _sc_epilogue(acc, rows0, out0, out1, cid, sid):
    plsc.subcore_barrier()
    row0 = sid * ROWS_PER_TILE
    for k in range(ROWS_PER_TILE // CHUNK):
        sl = pl.ds(row0 + k * CHUNK, CHUNK)
        pltpu.sync_copy(acc.at[sl], rows0)

        @pl.when(cid == 0)
        def _():
            pltpu.sync_copy(rows0, out0.at[sl])

        @pl.when(cid == 1)
        def _():
            pltpu.sync_copy(rows0, out1.at[sl])


def _sc_scratch():
    t = []
    for _ in range(NSETS):
        t.append(pltpu.VMEM((CHUNK,), jnp.int32))
    for _ in range(NSETS):
        t.append(pltpu.VMEM((CHUNK,), jnp.int32))
    for _ in range(NSETS):
        t.append(pltpu.VMEM((CHUNK, 128), jnp.float32))
    t.append(pltpu.VMEM_SHARED((NPAD, 128), jnp.float32))
    for _ in range(3 * NSETS):
        t.append(pltpu.SemaphoreType.DMA)
    return t


NCH_P = E // (NC * NS * CHUNK)
NCH_C = E // (NS * CHUNK)


@jax.jit
def _sc_pass_partial(table, src, dst):
    @functools.partial(
        pl.kernel, mesh=_vmesh(),
        out_type=[jax.ShapeDtypeStruct((NPAD, 128), jnp.float32),
                  jax.ShapeDtypeStruct((NPAD, 128), jnp.float32)],
        scratch_types=_sc_scratch(),
    )
    def k(table_h, src_h, dst_h, out0_h, out1_h, *scr):
        isrc = scr[0:NSETS]
        idst = scr[NSETS:2 * NSETS]
        rows = scr[2 * NSETS:3 * NSETS]
        acc = scr[3 * NSETS]
        sem_i = scr[3 * NSETS + 1:3 * NSETS + 1 + NSETS]
        sem_g = scr[3 * NSETS + 1 + NSETS:3 * NSETS + 1 + 2 * NSETS]
        sem_s = scr[3 * NSETS + 1 + 2 * NSETS:3 * NSETS + 1 + 3 * NSETS]
        cid = lax.axis_index("c")
        sid = lax.axis_index("s")
        w = cid * NS + sid
        _sc_prologue(acc, rows[0], sid)
        plsc.subcore_barrier()
        _scatter_chunks(table_h, src_h, dst_h, w * (E // (NC * NS)), acc,
                        isrc, idst, rows, sem_i, sem_g, sem_s, NCH_P)
        _sc_epilogue(acc, rows[0], out0_h, out1_h, cid, sid)

    return k(table, src, dst)


@jax.jit
def _sc_pass_colsplit(table_lo, table_hi, src, dst):
    @functools.partial(
        pl.kernel, mesh=_vmesh(),
        out_type=[jax.ShapeDtypeStruct((NPAD, 128), jnp.float32),
                  jax.ShapeDtypeStruct((NPAD, 128), jnp.float32)],
        scratch_types=_sc_scratch(),
    )
    def k(lo_h, hi_h, src_h, dst_h, out0_h, out1_h, *scr):
        isrc = scr[0:NSETS]
        idst = scr[NSETS:2 * NSETS]
        rows = scr[2 * NSETS:3 * NSETS]
        acc = scr[3 * NSETS]
        sem_i = scr[3 * NSETS + 1:3 * NSETS + 1 + NSETS]
        sem_g = scr[3 * NSETS + 1 + NSETS:3 * NSETS + 1 + 2 * NSETS]
        sem_s = scr[3 * NSETS + 1 + 2 * NSETS:3 * NSETS + 1 + 3 * NSETS]
        cid = lax.axis_index("c")
        sid = lax.axis_index("s")
        _sc_prologue(acc, rows[0], sid)
        plsc.subcore_barrier()
        base = sid * (E // NS)

        @pl.when(cid == 0)
        def _():
            _scatter_chunks(lo_h, src_h, dst_h, base, acc, isrc, idst,
                            rows, sem_i, sem_g, sem_s, NCH_C)

        @pl.when(cid == 1)
        def _():
            _scatter_chunks(hi_h, src_h, dst_h, base, acc, isrc, idst,
                            rows, sem_i, sem_g, sem_s, NCH_C)

        _sc_epilogue(acc, rows[0], out0_h, out1_h, cid, sid)

    return k(table_lo, table_hi, src, dst)


RBLK = 2000

_DOT = dict(preferred_element_type=jnp.float32, precision=lax.Precision.HIGHEST)


def _matvec_body(*refs, nh):
    it = iter(refs)
    h_refs = [next(it) for _ in range(nh)]
    w_refs = [next(it) for _ in range(nh)]
    b_ref = next(it)
    o_ref = next(it)
    acc = b_ref[...].astype(jnp.float32)
    for h_ref, w_ref in zip(h_refs, w_refs):
        acc = acc + jnp.dot(h_ref[...], w_ref[...], **_DOT)
    o_ref[...] = acc


def _tc_matvec(hs, ws, b):
    nh = len(hs)
    m = ws[0].shape[1]
    in_specs = [pl.BlockSpec((RBLK, h.shape[1]), lambda i: (i, 0))
                for h in hs]
    in_specs += [pl.BlockSpec(w.shape, lambda i: (0, 0)) for w in ws]
    in_specs.append(pl.BlockSpec((1, m), lambda i: (0, 0)))
    return pl.pallas_call(
        functools.partial(_matvec_body, nh=nh),
        grid=(N // RBLK,),
        in_specs=in_specs,
        out_specs=pl.BlockSpec((RBLK, m), lambda i: (i, 0)),
        out_shape=jax.ShapeDtypeStruct((N, m), jnp.float32),
    )(*hs, *ws, b)


def _gru_body(*refs, d, relu, nh, nout):
    it = iter(refs)
    sa_ref, sb_ref, gh_ref = next(it), next(it), next(it)
    h_refs = [next(it) for _ in range(nh)]
    wa_ref, wb_ref = next(it), next(it)
    bih_ref = next(it)
    o_refs = [next(it) for _ in range(nout)]

    gi = (jnp.dot(sa_ref[...], wa_ref[...], **_DOT)
          + jnp.dot(sb_ref[...], wb_ref[...], **_DOT) + bih_ref[...])
    gh = gh_ref[...]
    r = jax.nn.sigmoid(gi[:, :d] + gh[:, :d])
    z = jax.nn.sigmoid(gi[:, d:2 * d] + gh[:, d:2 * d])
    nn = jnp.tanh(gi[:, 2 * d:] + r * gh[:, 2 * d:])
    h = jnp.concatenate([h_ref[...] for h_ref in h_refs], axis=1)
    if h.shape[1] < d:
        h = jnp.concatenate(
            [h, jnp.zeros((h.shape[0], d - h.shape[1]), h.dtype)], axis=1)
    out = (1.0 - z) * nn + z * h
    if relu:
        out = jnp.maximum(out, 0.0)
    if nout == 1:
        o_refs[0][...] = out
    else:
        for i, o_ref in enumerate(o_refs):
            o_ref[...] = out[:, i * 128:(i + 1) * 128]


def _tc_gru(sa, sb, gh, hs, wca, wcb, bih, d, relu, nout):
    nh = len(hs)
    in_specs = [
        pl.BlockSpec((RBLK, 128), lambda i: (i, 0)),
        pl.BlockSpec((RBLK, 128), lambda i: (i, 0)),
        pl.BlockSpec((RBLK, 3 * d), lambda i: (i, 0)),
    ]
    for h in hs:
        in_specs.append(pl.BlockSpec((RBLK, h.shape[1]), lambda i: (i, 0)))
    in_specs.append(pl.BlockSpec((128, 3 * d), lambda i: (0, 0)))
    in_specs.append(pl.BlockSpec((128, 3 * d), lambda i: (0, 0)))
    in_specs.append(pl.BlockSpec((1, 3 * d), lambda i: (0, 0)))
    if nout == 1:
        out_specs = pl.BlockSpec((RBLK, d), lambda i: (i, 0))
        out_shape = jax.ShapeDtypeStruct((N, d), jnp.float32)
    else:
        out_specs = [pl.BlockSpec((RBLK, 128), lambda i: (i, 0))
                     for _ in range(nout)]
        out_shape = [jax.ShapeDtypeStruct((N, 128), jnp.float32)
                     for _ in range(nout)]
    return pl.pallas_call(
        functools.partial(_gru_body, d=d, relu=relu, nh=nh, nout=nout),
        grid=(N // RBLK,),
        in_specs=in_specs,
        out_specs=out_specs,
        out_shape=out_shape,
    )(sa, sb, gh, *hs, wca, wcb, bih)


def _segmax_body(xlo_ref, xhi_ref, b_ref, wf_ref, bf_ref, o_ref):
    b = b_ref[...]
    x = jnp.concatenate([xlo_ref[...], xhi_ref[...]], axis=1)

    def body(g, segs):
        v = jnp.where(b == g, x, -jnp.inf)
        seg = jnp.max(v, axis=0, keepdims=True)
        row_iota = lax.broadcasted_iota(jnp.int32, (G, 1), 0)
        return jnp.where(row_iota == g, seg, segs)

    segs = lax.fori_loop(0, G, body,
                         jnp.full((G, D2), -jnp.inf, jnp.float32))
    o_ref[...] = jnp.dot(segs, wf_ref[...], **_DOT) + bf_ref[...]


def _tc_segmax_head(xlo, xhi, batch2d, wf_pad, bf_pad):
    return pl.pallas_call(
        _segmax_body,
        grid=(1,),
        in_specs=[
            pl.BlockSpec((N, 128), lambda g: (0, 0)),
            pl.BlockSpec((N, 128), lambda g: (0, 0)),
            pl.BlockSpec((N, 1), lambda g: (0, 0)),
            pl.BlockSpec((D2, 128), lambda g: (0, 0)),
            pl.BlockSpec((1, 128), lambda g: (0, 0)),
        ],
        out_specs=pl.BlockSpec((G, 128), lambda g: (0, 0)),
        out_shape=jax.ShapeDtypeStruct((G, 128), jnp.float32),
    )(xlo, xhi, batch2d, wf_pad, bf_pad)


def _wc_body(w1_ref, wih1_ref, w2_ref, wih2_ref, o1a_ref, o1b_ref,
             o2a_ref, o2b_ref):
    wih1t = wih1_ref[...].T
    wih2t = wih2_ref[...].T
    o1a_ref[...] = jnp.dot(w1_ref[0], wih1t, **_DOT)
    o1b_ref[...] = jnp.dot(w1_ref[1], wih1t, **_DOT)
    o2a_ref[...] = jnp.dot(w2_ref[0, :128, :], wih2t, **_DOT)
    o2b_ref[...] = jnp.dot(w2_ref[1], wih2t, **_DOT)


def _tc_wc(weight1, Wih1, weight2, Wih2):
    return pl.pallas_call(
        _wc_body,
        grid=(1,),
        in_specs=[
            pl.BlockSpec((2, 128, 128), lambda i: (0, 0, 0)),
            pl.BlockSpec((384, 128), lambda i: (0, 0)),
            pl.BlockSpec((2, 256, 256), lambda i: (0, 0, 0)),
            pl.BlockSpec((768, 256), lambda i: (0, 0)),
        ],
        out_specs=[
            pl.BlockSpec((128, 384), lambda i: (0, 0)),
            pl.BlockSpec((128, 384), lambda i: (0, 0)),
            pl.BlockSpec((128, 768), lambda i: (0, 0)),
            pl.BlockSpec((256, 768), lambda i: (0, 0)),
        ],
        out_shape=[
            jax.ShapeDtypeStruct((128, 384), jnp.float32),
            jax.ShapeDtypeStruct((128, 384), jnp.float32),
            jax.ShapeDtypeStruct((128, 768), jnp.float32),
            jax.ShapeDtypeStruct((256, 768), jnp.float32),
        ],
    )(weight1, Wih1, weight2, Wih2)


def kernel(x, edge_index, batch, weight1, Wih1, Whh1, bih1, bhh1,
           weight2, Wih2, Whh2, bih2, bhh2, Wf, bf):
    src = edge_index[0].astype(jnp.int32)
    dst = edge_index[1].astype(jnp.int32)

    wc1_0, wc1_1, wc2_0, wc2_1 = _tc_wc(weight1, Wih1, weight2, Wih2)
    whhT1 = Whh1.T
    whhT2 = Whh2.T
    bih1r = bih1.reshape(1, -1)
    bhh1r = bhh1.reshape(1, -1)
    bih2r = bih2.reshape(1, -1)
    bhh2r = bhh2.reshape(1, -1)

    s1a, s1b = _sc_pass_partial(x, src, dst)
    gh1 = _tc_matvec([x], [whhT1], bhh1r)
    x1 = _tc_gru(s1a, s1b, gh1, [x], wc1_0, wc1_0, bih1r, D1, False, 1)

    s2a, s2b = _sc_pass_partial(x1, src, dst)
    gh2 = _tc_matvec([x1], [whhT1], bhh1r)
    y = _tc_gru(s2a, s2b, gh2, [x1], wc1_1, wc1_1, bih1r, D1, True, 1)

    s3a, s3b = _sc_pass_partial(y, src, dst)
    gh3 = _tc_matvec([y], [whhT2[:128, :]], bhh2r)
    x3lo, x3hi = _tc_gru(s3a, s3b, gh3, [y], wc2_0, wc2_0, bih2r,
                         D2, False, 2)

    s4a, s4b = _sc_pass_colsplit(x3lo, x3hi, src, dst)
    gh4 = _tc_matvec([x3lo, x3hi], [whhT2[:128, :], whhT2[128:, :]], bhh2r)
    x4lo, x4hi = _tc_gru(s4a, s4b, gh4, [x3lo, x3hi], wc2_1[:128, :],
                         wc2_1[128:, :], bih2r, D2, False, 2)

    batch2d = batch.astype(jnp.int32).reshape(N, 1)
    wf_pad = jnp.zeros((D2, 128), jnp.float32).at[:, :6].set(Wf.T)
    bf_pad = jnp.zeros((1, 128), jnp.float32).at[0, :6].set(bf)
    out = _tc_segmax_head(x4lo, x4hi, batch2d, wf_pad, bf_pad)
    return out[:, :6]
